# baseline (speedup 1.0000x reference)
import functools

import jax
import jax.numpy as jnp
from jax import lax
from jax.experimental import pallas as pl
from jax.experimental.pallas import tpu as pltpu
from jax.experimental.pallas import tpu_sc as plsc

_TOPK = 8
_NE = 64
_BLK = 512
_N = 8192
_NW = 32
_RPW = _N // _NW
_NEG = float("-inf")


def _matmul_kernel(x_ref, w_ref, b_ref, lt_ref):
    lt_ref[...] = jax.lax.dot_general(
        w_ref[...], x_ref[...], (((1,), (1,)), ((), ())),
        preferred_element_type=jnp.float32) + b_ref[...]


def _tc_logits_t(x, W, b):
    n_rows = x.shape[0]
    return pl.pallas_call(
        _matmul_kernel,
        grid=(n_rows // _BLK,),
        in_specs=[
            pl.BlockSpec((_BLK, x.shape[1]), lambda i: (i, 0)),
            pl.BlockSpec((_NE, x.shape[1]), lambda i: (0, 0)),
            pl.BlockSpec((_NE, _BLK), lambda i: (0, 0)),
        ],
        out_specs=pl.BlockSpec((_NE, _BLK), lambda i: (0, i)),
        out_shape=jax.ShapeDtypeStruct((_NE, n_rows), jnp.float32),
        compiler_params=pltpu.CompilerParams(
            dimension_semantics=("parallel",)),
    )(x, W, jnp.broadcast_to(b[:, None], (_NE, _BLK)))


@functools.partial(
    pl.kernel,
    mesh=plsc.VectorSubcoreMesh(core_axis_name="c", subcore_axis_name="s"),
    out_type=[
        jax.ShapeDtypeStruct((_NE, _N), jnp.float32),
        jax.ShapeDtypeStruct((_TOPK, _N), jnp.int32),
    ],
    scratch_types=[
        pltpu.VMEM((_NE, _RPW), jnp.float32),
        pltpu.VMEM((_NE, _RPW), jnp.float32),
        pltpu.VMEM((_TOPK, _RPW), jnp.int32),
    ],
)
def _sc_router(lt_hbm, router_hbm, idx_hbm, lt_v, out_v, idx_v):
    wid = lax.axis_index("s") * 2 + lax.axis_index("c")
    base = wid * _RPW
    pltpu.sync_copy(lt_hbm.at[:, pl.ds(base, _RPW)], lt_v)

    def group_body(j, carry):
        r0 = pl.multiple_of(j * 16, 16)
        v = [lt_v[e, pl.ds(r0, 16)] for e in range(_NE)]
        ms, mis = [], []
        for _ in range(_TOPK):
            m = v[0]
            for e in range(1, _NE):
                m = jnp.maximum(m, v[e])
            acc = jnp.zeros((16,), jnp.int32)
            for e in range(_NE - 1, -1, -1):
                is_m = v[e] == m
                acc = jnp.where(is_m, jnp.int32(e), acc)
                v[e] = jnp.where(is_m, _NEG, v[e])
            ms.append(m)
            mis.append(acc)
        m0 = ms[0]
        exs = [jnp.exp(m - m0) for m in ms]
        denom = exs[0]
        for k in range(1, _TOPK):
            denom = denom + exs[k]
        rnorm = 1.0 / denom
        thresh = ms[_TOPK - 1]
        for e in range(_NE):
            v2 = lt_v[e, pl.ds(r0, 16)]
            out_v[e, pl.ds(r0, 16)] = jnp.where(
                v2 >= thresh, jnp.exp(v2 - m0) * rnorm, jnp.float32(0.0))
        for k in range(_TOPK):
            idx_v[k, pl.ds(r0, 16)] = mis[k]
        return carry

    lax.fori_loop(0, _RPW // 16, group_body, 0)

    pltpu.sync_copy(out_v, router_hbm.at[:, pl.ds(base, _RPW)])
    pltpu.sync_copy(idx_v, idx_hbm.at[:, pl.ds(base, _RPW)])


@jax.jit
def kernel(x, W, b):
    logits_t = _tc_logits_t(x, W, b)
    router_t, idx_t = _sc_router(logits_t)
    return (router_t.T, idx_t.T)


# speedup vs baseline: 1.1209x; 1.1209x over previous
import functools

import jax
import jax.numpy as jnp
from jax import lax
from jax.experimental import pallas as pl
from jax.experimental.pallas import tpu as pltpu
from jax.experimental.pallas import tpu_sc as plsc

_TOPK = 8
_NE = 64
_BLK = 512
_N = 8192
_NW = 32
_RPW = _N // _NW
_NEG = float("-inf")


def _matmul_kernel(x_ref, w_ref, b_ref, lt_ref):
    lt_ref[...] = jax.lax.dot_general(
        w_ref[...], x_ref[...], (((1,), (1,)), ((), ())),
        preferred_element_type=jnp.float32) + b_ref[...]


def _tc_logits_t(x, W, b):
    n_rows = x.shape[0]
    return pl.pallas_call(
        _matmul_kernel,
        grid=(n_rows // _BLK,),
        in_specs=[
            pl.BlockSpec((_BLK, x.shape[1]), lambda i: (i, 0)),
            pl.BlockSpec((_NE, x.shape[1]), lambda i: (0, 0)),
            pl.BlockSpec((_NE, _BLK), lambda i: (0, 0)),
        ],
        out_specs=pl.BlockSpec((_NE, _BLK), lambda i: (0, i)),
        out_shape=jax.ShapeDtypeStruct((_NE, n_rows), jnp.float32),
        compiler_params=pltpu.CompilerParams(
            dimension_semantics=("parallel",)),
    )(x, W, jnp.broadcast_to(b[:, None], (_NE, _BLK)))


@functools.partial(
    pl.kernel,
    mesh=plsc.VectorSubcoreMesh(core_axis_name="c", subcore_axis_name="s"),
    out_type=[
        jax.ShapeDtypeStruct((_NE, _N), jnp.float32),
        jax.ShapeDtypeStruct((_TOPK, _N), jnp.int32),
    ],
    scratch_types=[
        pltpu.VMEM((_NE, _RPW), jnp.float32),
        pltpu.VMEM((_NE, _RPW), jnp.float32),
        pltpu.VMEM((_TOPK, _RPW), jnp.int32),
    ],
)
def _sc_router(lt_hbm, router_hbm, idx_hbm, lt_v, out_v, idx_v):
    wid = lax.axis_index("s") * 2 + lax.axis_index("c")
    base = wid * _RPW
    pltpu.sync_copy(lt_hbm.at[:, pl.ds(base, _RPW)], lt_v)

    net8 = ((0, 1), (2, 3), (4, 5), (6, 7), (0, 2), (1, 3), (4, 6),
            (5, 7), (1, 2), (5, 6), (0, 4), (3, 7), (1, 5), (2, 6),
            (1, 4), (3, 6), (2, 4), (3, 5), (3, 4))
    clean8 = ((0, 4), (1, 5), (2, 6), (3, 7), (0, 2), (1, 3), (4, 6),
              (5, 7), (0, 1), (2, 3), (4, 5), (6, 7))

    def ce(vals, idxs, i, j):
        take = vals[i] >= vals[j]
        vi = jnp.where(take, vals[i], vals[j])
        vj = jnp.where(take, vals[j], vals[i])
        ii = jnp.where(take, idxs[i], idxs[j])
        ij = jnp.where(take, idxs[j], idxs[i])
        vals[i], vals[j], idxs[i], idxs[j] = vi, vj, ii, ij

    def group_body(j, carry):
        r0 = pl.multiple_of(j * 16, 16)

        def sorted_chunk(c):
            vals = [lt_v[8 * c + u, pl.ds(r0, 16)] for u in range(8)]
            idxs = [jnp.full((16,), 8 * c + u, jnp.int32) for u in range(8)]
            for a, b_ in net8:
                ce(vals, idxs, a, b_)
            return vals, idxs

        run_v, run_i = sorted_chunk(0)
        for c in range(1, _NE // 8):
            cv, ci = sorted_chunk(c)
            mv, mi = [], []
            for u in range(8):
                take = run_v[u] >= cv[7 - u]
                mv.append(jnp.where(take, run_v[u], cv[7 - u]))
                mi.append(jnp.where(take, run_i[u], ci[7 - u]))
            for a, b_ in clean8:
                ce(mv, mi, a, b_)
            run_v, run_i = mv, mi

        m0 = run_v[0]
        exs = [jnp.exp(v - m0) for v in run_v]
        denom = exs[0]
        for k in range(1, _TOPK):
            denom = denom + exs[k]
        rnorm = 1.0 / denom
        thresh = run_v[_TOPK - 1]
        for e in range(_NE):
            v2 = lt_v[e, pl.ds(r0, 16)]
            out_v[e, pl.ds(r0, 16)] = jnp.where(
                v2 >= thresh, jnp.exp(v2 - m0) * rnorm, jnp.float32(0.0))
        for k in range(_TOPK):
            idx_v[k, pl.ds(r0, 16)] = run_i[k]
        return carry

    lax.fori_loop(0, _RPW // 16, group_body, 0)

    pltpu.sync_copy(out_v, router_hbm.at[:, pl.ds(base, _RPW)])
    pltpu.sync_copy(idx_v, idx_hbm.at[:, pl.ds(base, _RPW)])


@jax.jit
def kernel(x, W, b):
    logits_t = _tc_logits_t(x, W, b)
    router_t, idx_t = _sc_router(logits_t)
    return (router_t.T, idx_t.T)


# speedup vs baseline: 3.3816x; 3.0169x over previous
import jax
import jax.numpy as jnp
from jax.experimental import pallas as pl
from jax.experimental.pallas import tpu as pltpu

_TOPK = 8
_NE = 64
_BLK = 1024


def _router_kernel(x_ref, w_ref, b_ref, router_ref, idx_ref):
    x = x_ref[...]
    w = w_ref[...]
    logits_t = jax.lax.dot_general(
        w, x, (((1,), (1,)), ((), ())),
        preferred_element_type=jnp.float32)
    logits_t = logits_t + b_ref[...]

    raw = jax.lax.bitcast_convert_type(logits_t, jnp.int32)
    mono = raw ^ jax.lax.shift_right_logical(
        jax.lax.shift_right_arithmetic(raw, 31), 1)
    iota_e = jax.lax.broadcasted_iota(jnp.int32, logits_t.shape, 0)

    def argmax_rows(v, i):
        while v.shape[0] > 1:
            h = v.shape[0] // 2
            take_a = v[:h] >= v[h:]
            v = jnp.where(take_a, v[:h], v[h:])
            i = jnp.where(take_a, i[:h], i[h:])
        return v, i

    work = mono
    idx_rows = []
    m0 = mlast = None
    for k in range(_TOPK):
        m, mi = argmax_rows(work, iota_e)
        if k == 0:
            m0 = m
        mlast = m
        if k + 1 < _TOPK:
            sel = iota_e == mi
            work = jnp.where(sel, jnp.int32(-2**31), work)
        idx_rows.append(mi)
    mask = mono >= mlast

    maxv = jax.lax.bitcast_convert_type(
        m0 ^ jax.lax.shift_right_logical(
            jax.lax.shift_right_arithmetic(m0, 31), 1), jnp.float32)
    ex = jnp.where(mask, jnp.exp(logits_t - maxv), jnp.float32(0.0))
    denom = jnp.sum(ex, axis=0, keepdims=True)
    router_t = ex / denom
    router_ref[...] = router_t
    for k in range(_TOPK):
        idx_ref[k, :] = idx_rows[k][0, :]


@jax.jit
def kernel(x, W, b):
    n_rows = x.shape[0]
    grid = (n_rows // _BLK,)
    router, idx = pl.pallas_call(
        _router_kernel,
        grid=grid,
        in_specs=[
            pl.BlockSpec((_BLK, x.shape[1]), lambda i: (i, 0)),
            pl.BlockSpec((_NE, x.shape[1]), lambda i: (0, 0)),
            pl.BlockSpec((_NE, _BLK), lambda i: (0, 0)),
        ],
        out_specs=[
            pl.BlockSpec((_NE, _BLK), lambda i: (0, i)),
            pl.BlockSpec((_TOPK, _BLK), lambda i: (0, i)),
        ],
        out_shape=[
            jax.ShapeDtypeStruct((_NE, n_rows), jnp.float32),
            jax.ShapeDtypeStruct((_TOPK, n_rows), jnp.int32),
        ],
        compiler_params=pltpu.CompilerParams(
            dimension_semantics=("parallel",)),
    )(x, W, jnp.broadcast_to(b[:, None], (_NE, _BLK)))
    return router.T, idx.T
